# baseline (speedup 1.0000x reference)
import functools

import jax
import jax.numpy as jnp
from jax import lax
from jax.experimental import pallas as pl
from jax.experimental.pallas import tpu as pltpu
from jax.experimental.pallas import tpu_sc as plsc

B = 16384
NCF = 13
NS = 26
V = 100000
D = 32
OUT_W = NCF + NS * D
CAT_W = NS * D

_info = plsc.get_sparse_core_info()
NCORES = _info.num_cores
NSUB = _info.num_subcores
LANES = _info.num_lanes
NW = NCORES * NSUB
RPW = B // NW

CB = 16
NCH = RPW // CB
NIDX = CB * NS
GL = 32
NG = NIDX // GL
OROWS = NIDX * D // 128

_mesh = plsc.VectorSubcoreMesh(core_axis_name="c", subcore_axis_name="s")


@functools.partial(
    pl.kernel,
    mesh=_mesh,
    compiler_params=pltpu.CompilerParams(
        use_tc_tiling_on_sc=True, needs_layout_passes=False),
    out_type=jax.ShapeDtypeStruct((B * CAT_W // 128, 128), jnp.float32),
    scratch_types=[
        pltpu.VMEM((CB, NS), jnp.int32),
        pltpu.VMEM((NG, GL), jnp.int32),
        pltpu.VMEM((NIDX,), jnp.int32),
        pltpu.VMEM((NIDX,), jnp.int32),
        pltpu.VMEM((NIDX,), jnp.int32),
        pltpu.VMEM((NIDX,), jnp.int32),
        pltpu.VMEM((NIDX, 128), jnp.float32),
        pltpu.VMEM((OROWS, 128), jnp.float32),
        pltpu.SemaphoreType.DMA,
    ],
)
def _gather(cat_hbm, tab_hbm, out_hbm,
            catv, idxf, q4b, offp, gr, gc, g128, obuf, gsem):
    wid = lax.axis_index("s") * NCORES + lax.axis_index("c")
    row0 = wid * RPW
    iota = lax.iota(jnp.int32, LANES)

    for k in range(NIDX // LANES):
        p = iota + k * LANES
        s = p - (p // NS) * NS
        offp[pl.ds(k * LANES, LANES)] = s * (V // 4)
        gr[pl.ds(k * LANES, LANES)] = p // NS
        gc[pl.ds(k * LANES, LANES)] = s

    def chunk_body(g, carry):
        base = pl.multiple_of(row0 + g * CB, CB)
        pltpu.sync_copy(cat_hbm.at[pl.ds(base, CB), :], catv)
        for k in range(NIDX // LANES):
            sl = pl.ds(k * LANES, LANES)
            raw = plsc.load_gather(catv, [gr[sl], gc[sl]])
            idxf[k // 2, pl.ds((k % 2) * LANES, LANES)] = (
                (raw >> 2) + offp[sl])
            q4b[sl] = raw - ((raw >> 2) << 2)
        cps = [
            pltpu.async_copy(
                tab_hbm.at[idxf.at[j]],
                g128.at[pl.ds(j * GL, GL), :],
                gsem)
            for j in range(NG)
        ]
        for cp in cps:
            cp.wait()

        def ent_body(r, carry2):
            q4 = plsc.load_gather(q4b, [iota * 0 + r])
            for h in range(2):
                v = plsc.load_gather(
                    g128, [iota * 0 + r, iota + (h * LANES) + (q4 << 5)])
                w = r * D + h * LANES
                plsc.store_scatter(
                    obuf, [iota * 0 + (w >> 7), iota + (w - ((w >> 7) << 7))],
                    v)
            return carry2

        lax.fori_loop(0, NIDX, ent_body, 0)
        erow = pl.multiple_of(base * CAT_W // 128, CB * CAT_W // 128)
        pltpu.sync_copy(obuf, out_hbm.at[pl.ds(erow, OROWS), :])
        return carry

    lax.fori_loop(0, NCH, chunk_body, 0)


def kernel(x_continuous, x_categorical, tables):
    cat = x_categorical.astype(jnp.int32)
    tab = tables.reshape(NS * V * D // 128, 128)
    emb = _gather(cat, tab)
    return jnp.concatenate([x_continuous, emb.reshape(B, CAT_W)], axis=-1)


# speedup vs baseline: 1.2200x; 1.2200x over previous
import functools

import jax
import jax.numpy as jnp
from jax import lax
from jax.experimental import pallas as pl
from jax.experimental.pallas import tpu as pltpu
from jax.experimental.pallas import tpu_sc as plsc

B = 16384
NCF = 13
NS = 26
V = 100000
D = 32
OUT_W = NCF + NS * D
CAT_W = NS * D

_info = plsc.get_sparse_core_info()
NCORES = _info.num_cores
NSUB = _info.num_subcores
LANES = _info.num_lanes
NW = NCORES * NSUB
RPW = B // NW

CB = 64
NCH = RPW // CB
NIDX = CB * NS
GL = 128
NG = NIDX // GL

_mesh = plsc.VectorSubcoreMesh(core_axis_name="c", subcore_axis_name="s")


@functools.partial(
    pl.kernel,
    mesh=_mesh,
    compiler_params=pltpu.CompilerParams(
        use_tc_tiling_on_sc=False, needs_layout_passes=False),
    out_type=jax.ShapeDtypeStruct((B * NS, D), jnp.float32),
    scratch_types=[
        pltpu.VMEM((CB, NS), jnp.int32),
        pltpu.VMEM((NG, GL), jnp.int32),
        pltpu.VMEM((NIDX,), jnp.int32),
        pltpu.VMEM((NIDX,), jnp.int32),
        pltpu.VMEM((NIDX,), jnp.int32),
        pltpu.VMEM((NIDX, D), jnp.float32),
        pltpu.SemaphoreType.DMA,
    ],
)
def _gather(cat_hbm, tab_hbm, out_hbm, catv, idxf, offp, gr, gc, gbuf, gsem):
    wid = lax.axis_index("s") * NCORES + lax.axis_index("c")
    row0 = wid * RPW
    iota = lax.iota(jnp.int32, LANES)

    for k in range(NIDX // LANES):
        p = iota + k * LANES
        s = p - (p // NS) * NS
        offp[pl.ds(k * LANES, LANES)] = s * V
        gr[pl.ds(k * LANES, LANES)] = p // NS
        gc[pl.ds(k * LANES, LANES)] = s

    def chunk_body(g, carry):
        base = row0 + g * CB
        pltpu.sync_copy(cat_hbm.at[pl.ds(base, CB), :], catv)
        for k in range(NIDX // LANES):
            sl = pl.ds(k * LANES, LANES)
            vals = plsc.load_gather(catv, [gr[sl], gc[sl]])
            idxf[k // 8, pl.ds((k % 8) * LANES, LANES)] = vals + offp[sl]
        cps = [
            pltpu.async_copy(
                tab_hbm.at[idxf.at[j]],
                gbuf.at[pl.ds(j * GL, GL), :],
                gsem)
            for j in range(NG)
        ]
        for cp in cps:
            cp.wait()
        pltpu.sync_copy(gbuf, out_hbm.at[pl.ds(base * NS, NIDX), :])
        return carry

    lax.fori_loop(0, NCH, chunk_body, 0)


def kernel(x_continuous, x_categorical, tables):
    cat = x_categorical.astype(jnp.int32)
    tab = tables.reshape(NS * V, D)
    emb = _gather(cat, tab)
    return jnp.concatenate([x_continuous, emb.reshape(B, CAT_W)], axis=-1)
